# baseline (speedup 1.0000x reference)
import functools

import jax
import jax.numpy as jnp
from jax import lax
from jax.experimental import pallas as pl
from jax.experimental.pallas import tpu as pltpu
from jax.experimental.pallas import tpu_sc as plsc

N_EMBD = 768
EXPERT_DIM = 1536
NUM_EXPERTS = 64
TOP_K = 2

N_TOK = 2048
BT = 128
MAX_TILES = 96
ROWS = MAX_TILES * BT
NW = 32
TPW = N_TOK // NW
LANES = 16

_mesh = plsc.VectorSubcoreMesh(core_axis_name="c", subcore_axis_name="s")


def _router_body(x_ref, wg_ref, w0_ref, w1_ref, oh0_ref, oh1_ref, xb_ref):
    x = x_ref[...]
    wg = wg_ref[...]
    xb = x.astype(jnp.bfloat16)
    xb_ref[...] = xb
    logits = lax.dot_general(
        xb, wg.astype(jnp.bfloat16),
        (((1,), (1,)), ((), ())),
        preferred_element_type=jnp.float32)
    p = jax.nn.softmax(logits, axis=-1)
    m1 = jnp.max(p, axis=-1)
    a1 = jnp.argmax(p, axis=-1).astype(jnp.int32)
    e_iota = lax.broadcasted_iota(jnp.int32, p.shape, 1)
    p2 = jnp.where(e_iota == a1[:, None], -jnp.inf, p)
    m2 = jnp.max(p2, axis=-1)
    a2 = jnp.argmax(p2, axis=-1).astype(jnp.int32)
    denom = m1 + m2 + 1e-8
    w0_ref[...] = (m1 / denom)[:, None]
    w1_ref[...] = (m2 / denom)[:, None]
    oh0_ref[...] = (e_iota == a1[:, None]).astype(jnp.bfloat16)
    oh1_ref[...] = (e_iota == a2[:, None]).astype(jnp.bfloat16)


def _router(flat_x, Wg):
    return pl.pallas_call(
        _router_body,
        out_shape=[
            jax.ShapeDtypeStruct((N_TOK, 1), jnp.float32),
            jax.ShapeDtypeStruct((N_TOK, 1), jnp.float32),
            jax.ShapeDtypeStruct((N_TOK, NUM_EXPERTS), jnp.bfloat16),
            jax.ShapeDtypeStruct((N_TOK, NUM_EXPERTS), jnp.bfloat16),
            jax.ShapeDtypeStruct((N_TOK, N_EMBD), jnp.bfloat16),
        ],
    )(flat_x, Wg)


def _plan_body(oh0_ref, oh1_ref, oh0b_ref, oh1b_ref,
               pos0_ref, pos1_ref, te_ref):
    b = pl.program_id(0)
    oh0 = oh0_ref[...]
    oh1 = oh1_ref[...]
    tot0 = jnp.sum(oh0.astype(jnp.float32), axis=0)
    cnt = tot0 + jnp.sum(oh1.astype(jnp.float32), axis=0)
    pc = jnp.floor((cnt + (BT - 1)) / BT) * BT
    ei = lax.broadcasted_iota(jnp.int32, (NUM_EXPERTS, NUM_EXPERTS), 0)
    ej = lax.broadcasted_iota(jnp.int32, (NUM_EXPERTS, NUM_EXPERTS), 1)
    trilE = (ej < ei).astype(jnp.bfloat16)
    base = lax.dot_general(
        trilE, pc.astype(jnp.bfloat16), (((1,), (0,)), ((), ())),
        preferred_element_type=jnp.float32)

    ri = lax.broadcasted_iota(jnp.int32, (BT, N_TOK), 0)
    cj = lax.broadcasted_iota(jnp.int32, (BT, N_TOK), 1)
    tril = (cj < b * BT + ri).astype(jnp.bfloat16)
    c0 = lax.dot_general(
        tril, oh0, (((1,), (0,)), ((), ())),
        preferred_element_type=jnp.float32)
    c1 = lax.dot_general(
        tril, oh1, (((1,), (0,)), ((), ())),
        preferred_element_type=jnp.float32)
    ohb0 = oh0b_ref[...].astype(jnp.float32)
    ohb1 = oh1b_ref[...].astype(jnp.float32)
    pos0 = jnp.sum((c0 + base[None, :]) * ohb0, axis=1)
    pos1 = jnp.sum((c1 + tot0[None, :] + base[None, :]) * ohb1, axis=1)
    pos0_ref[...] = pos0.astype(jnp.int32)[:, None]
    pos1_ref[...] = pos1.astype(jnp.int32)[:, None]

    @pl.when(b == 0)
    def _():
        bt_tile = base / BT
        ti = lax.broadcasted_iota(
            jnp.int32, (MAX_TILES, NUM_EXPERTS), 0).astype(jnp.float32)
        te = jnp.sum((bt_tile[None, :] <= ti).astype(jnp.float32),
                     axis=1) - 1.0
        te_ref[...] = te.astype(jnp.int32)[:, None]


def _plan(oh0, oh1):
    nblk = N_TOK // BT
    return pl.pallas_call(
        _plan_body,
        grid=(nblk,),
        in_specs=[
            pl.BlockSpec((N_TOK, NUM_EXPERTS), lambda b: (0, 0)),
            pl.BlockSpec((N_TOK, NUM_EXPERTS), lambda b: (0, 0)),
            pl.BlockSpec((BT, NUM_EXPERTS), lambda b: (b, 0)),
            pl.BlockSpec((BT, NUM_EXPERTS), lambda b: (b, 0)),
        ],
        out_specs=[
            pl.BlockSpec((BT, 1), lambda b: (b, 0)),
            pl.BlockSpec((BT, 1), lambda b: (b, 0)),
            pl.BlockSpec((MAX_TILES, 1), lambda b: (0, 0)),
        ],
        out_shape=[
            jax.ShapeDtypeStruct((N_TOK, 1), jnp.int32),
            jax.ShapeDtypeStruct((N_TOK, 1), jnp.int32),
            jax.ShapeDtypeStruct((MAX_TILES, 1), jnp.int32),
        ],
    )(oh0, oh1, oh0, oh1)


def _scatter_body(x_hbm, pos0_hbm, pos1_hbm, xg_hbm, pidx0, pidx1, xv, sem):
    c = lax.axis_index("c")
    s = lax.axis_index("s")
    g = c * 16 + s
    base = g * TPW
    pltpu.sync_copy(pos0_hbm.at[pl.ds(base, TPW)], pidx0)
    pltpu.sync_copy(pos1_hbm.at[pl.ds(base, TPW)], pidx1)
    pltpu.sync_copy(x_hbm.at[pl.ds(base, TPW), :], xv)
    pltpu.async_copy(xv, xg_hbm.at[pidx0], sem).wait()
    pltpu.async_copy(xv, xg_hbm.at[pidx1], sem).wait()


@functools.partial(
    pl.kernel, mesh=_mesh,
    out_type=[jax.ShapeDtypeStruct((ROWS, N_EMBD // 2), jnp.float32)],
    scratch_types=[
        pltpu.VMEM((TPW,), jnp.int32),
        pltpu.VMEM((TPW,), jnp.int32),
        pltpu.VMEM((TPW, N_EMBD // 2), jnp.float32),
        pltpu.SemaphoreType.DMA,
    ],
)
def _scatter(x_hbm, pos0_hbm, pos1_hbm, xg_hbm, *scratch):
    _scatter_body(x_hbm, pos0_hbm, pos1_hbm, xg_hbm, *scratch)


def _gemm_body(te_ref, xg_ref, wfc_ref, wpr_ref, yo_ref):
    xb = xg_ref[...]
    wfc = wfc_ref[0].astype(jnp.bfloat16)
    h = lax.dot_general(
        xb, wfc, (((1,), (1,)), ((), ())),
        preferred_element_type=jnp.float32)
    h = jnp.square(jnp.maximum(h, 0.0)).astype(jnp.bfloat16)
    wpr = wpr_ref[0].astype(jnp.bfloat16)
    yo_ref[...] = lax.dot_general(
        h, wpr, (((1,), (1,)), ((), ())),
        preferred_element_type=jnp.float32).astype(jnp.bfloat16)


def _gemm(te, xg, W_fc, W_proj):
    grid_spec = pltpu.PrefetchScalarGridSpec(
        num_scalar_prefetch=1,
        grid=(MAX_TILES,),
        in_specs=[
            pl.BlockSpec((BT, N_EMBD), lambda t, te_ref: (t, 0)),
            pl.BlockSpec((1, EXPERT_DIM, N_EMBD),
                         lambda t, te_ref: (te_ref[t], 0, 0)),
            pl.BlockSpec((1, N_EMBD, EXPERT_DIM),
                         lambda t, te_ref: (te_ref[t], 0, 0)),
        ],
        out_specs=pl.BlockSpec((BT, N_EMBD), lambda t, te_ref: (t, 0)),
    )
    return pl.pallas_call(
        _gemm_body,
        grid_spec=grid_spec,
        out_shape=jax.ShapeDtypeStruct((ROWS, N_EMBD), jnp.bfloat16),
    )(te, xg, W_fc, W_proj)


def _gather_body(yo_hbm, pos0_hbm, pos1_hbm, y0_hbm, y1_hbm,
                 pidx0, pidx1, buf0, buf1, sem):
    c = lax.axis_index("c")
    s = lax.axis_index("s")
    g = c * 16 + s
    base = g * TPW
    pltpu.sync_copy(pos0_hbm.at[pl.ds(base, TPW)], pidx0)
    pltpu.sync_copy(pos1_hbm.at[pl.ds(base, TPW)], pidx1)
    pltpu.async_copy(yo_hbm.at[pidx0], buf0, sem).wait()
    pltpu.async_copy(yo_hbm.at[pidx1], buf1, sem).wait()
    pltpu.sync_copy(buf0, y0_hbm.at[pl.ds(base, TPW), :])
    pltpu.sync_copy(buf1, y1_hbm.at[pl.ds(base, TPW), :])


@functools.partial(
    pl.kernel, mesh=_mesh,
    out_type=[
        jax.ShapeDtypeStruct((N_TOK, N_EMBD // 2), jnp.float32),
        jax.ShapeDtypeStruct((N_TOK, N_EMBD // 2), jnp.float32),
    ],
    scratch_types=[
        pltpu.VMEM((TPW,), jnp.int32),
        pltpu.VMEM((TPW,), jnp.int32),
        pltpu.VMEM((TPW, N_EMBD // 2), jnp.float32),
        pltpu.VMEM((TPW, N_EMBD // 2), jnp.float32),
        pltpu.SemaphoreType.DMA,
    ],
)
def _gather(yo_hbm, pos0_hbm, pos1_hbm, y0_hbm, y1_hbm, *scratch):
    _gather_body(yo_hbm, pos0_hbm, pos1_hbm, y0_hbm, y1_hbm, *scratch)


def _combine_body(y0_ref, y1_ref, w0_ref, w1_ref, out_ref):
    out_ref[...] = (y0_ref[...].astype(jnp.float32) * w0_ref[...]
                    + y1_ref[...].astype(jnp.float32) * w1_ref[...])


def _combine(y0, y1, w0, w1):
    return pl.pallas_call(
        _combine_body,
        out_shape=jax.ShapeDtypeStruct((N_TOK, N_EMBD), jnp.float32),
    )(y0, y1, w0, w1)


def _as_words(a):
    return lax.bitcast_convert_type(
        a.reshape(*a.shape[:-1], a.shape[-1] // 2, 2), jnp.float32)


def _as_bf16(a):
    return lax.bitcast_convert_type(a, jnp.bfloat16).reshape(
        *a.shape[:-1], a.shape[-1] * 2)


def kernel(x, Wg, W_fc, W_proj):
    B, T, D = x.shape
    flat_x = x.reshape(N_TOK, D)
    w0, w1, oh0, oh1, xb = _router(flat_x, Wg)
    pos0, pos1, te = _plan(oh0, oh1)
    xg_w, = _scatter(_as_words(xb), pos0.reshape(N_TOK), pos1.reshape(N_TOK))
    yo = _gemm(te.reshape(MAX_TILES), _as_bf16(xg_w), W_fc, W_proj)
    y0_w, y1_w = _gather(
        _as_words(yo), pos0.reshape(N_TOK), pos1.reshape(N_TOK))
    out = _combine(_as_bf16(y0_w), _as_bf16(y1_w), w0, w1)
    return out.reshape(B, T, D)


# speedup vs baseline: 2.5290x; 2.5290x over previous
import functools

import jax
import jax.numpy as jnp
from jax import lax
from jax.experimental import pallas as pl
from jax.experimental.pallas import tpu as pltpu
from jax.experimental.pallas import tpu_sc as plsc

N_EMBD = 768
EXPERT_DIM = 1536
NUM_EXPERTS = 64
TOP_K = 2

N_TOK = 2048
BT = 128
MAX_TILES = 96
ROWS = MAX_TILES * BT
NW = 32
TPW = N_TOK // NW
LANES = 16

_mesh = plsc.VectorSubcoreMesh(core_axis_name="c", subcore_axis_name="s")


def _router_body(x_ref, wg_ref, w0_ref, w1_ref, oh0_ref, oh1_ref):
    x = x_ref[...]
    wg = wg_ref[...]
    logits = lax.dot_general(
        x.astype(jnp.bfloat16), wg.astype(jnp.bfloat16),
        (((1,), (1,)), ((), ())),
        preferred_element_type=jnp.float32)
    p = jax.nn.softmax(logits, axis=-1)
    m1 = jnp.max(p, axis=-1)
    a1 = jnp.argmax(p, axis=-1).astype(jnp.int32)
    e_iota = lax.broadcasted_iota(jnp.int32, p.shape, 1)
    p2 = jnp.where(e_iota == a1[:, None], -jnp.inf, p)
    m2 = jnp.max(p2, axis=-1)
    a2 = jnp.argmax(p2, axis=-1).astype(jnp.int32)
    denom = m1 + m2 + 1e-8
    w0_ref[...] = (m1 / denom)[:, None]
    w1_ref[...] = (m2 / denom)[:, None]
    oh0_ref[...] = (e_iota == a1[:, None]).astype(jnp.bfloat16)
    oh1_ref[...] = (e_iota == a2[:, None]).astype(jnp.bfloat16)


def _router(flat_x, Wg):
    return pl.pallas_call(
        _router_body,
        out_shape=[
            jax.ShapeDtypeStruct((N_TOK, 1), jnp.float32),
            jax.ShapeDtypeStruct((N_TOK, 1), jnp.float32),
            jax.ShapeDtypeStruct((N_TOK, NUM_EXPERTS), jnp.bfloat16),
            jax.ShapeDtypeStruct((N_TOK, NUM_EXPERTS), jnp.bfloat16),
        ],
    )(flat_x, Wg)


def _plan_body(oh0_ref, oh1_ref, oh0b_ref, oh1b_ref,
               pos0_ref, pos1_ref, te_ref):
    b = pl.program_id(0)
    oh0 = oh0_ref[...]
    oh1 = oh1_ref[...]
    tot0 = jnp.sum(oh0.astype(jnp.float32), axis=0)
    cnt = tot0 + jnp.sum(oh1.astype(jnp.float32), axis=0)
    pc = jnp.floor((cnt + (BT - 1)) / BT) * BT
    ei = lax.broadcasted_iota(jnp.int32, (NUM_EXPERTS, NUM_EXPERTS), 0)
    ej = lax.broadcasted_iota(jnp.int32, (NUM_EXPERTS, NUM_EXPERTS), 1)
    trilE = (ej < ei).astype(jnp.bfloat16)
    base = lax.dot_general(
        trilE, pc.astype(jnp.bfloat16), (((1,), (0,)), ((), ())),
        preferred_element_type=jnp.float32)

    ri = lax.broadcasted_iota(jnp.int32, (BT, N_TOK), 0)
    cj = lax.broadcasted_iota(jnp.int32, (BT, N_TOK), 1)
    tril = (cj < b * BT + ri).astype(jnp.bfloat16)
    c0 = lax.dot_general(
        tril, oh0, (((1,), (0,)), ((), ())),
        preferred_element_type=jnp.float32)
    c1 = lax.dot_general(
        tril, oh1, (((1,), (0,)), ((), ())),
        preferred_element_type=jnp.float32)
    ohb0 = oh0b_ref[...].astype(jnp.float32)
    ohb1 = oh1b_ref[...].astype(jnp.float32)
    pos0 = jnp.sum((c0 + base[None, :]) * ohb0, axis=1)
    pos1 = jnp.sum((c1 + tot0[None, :] + base[None, :]) * ohb1, axis=1)
    pos0_ref[...] = pos0.astype(jnp.int32)[:, None]
    pos1_ref[...] = pos1.astype(jnp.int32)[:, None]

    @pl.when(b == 0)
    def _():
        bt_tile = base / BT
        ti = lax.broadcasted_iota(
            jnp.int32, (MAX_TILES, NUM_EXPERTS), 0).astype(jnp.float32)
        te = jnp.sum((bt_tile[None, :] <= ti).astype(jnp.float32),
                     axis=1) - 1.0
        te_ref[...] = te.astype(jnp.int32)[:, None]


def _plan(oh0, oh1):
    nblk = N_TOK // BT
    return pl.pallas_call(
        _plan_body,
        grid=(nblk,),
        in_specs=[
            pl.BlockSpec((N_TOK, NUM_EXPERTS), lambda b: (0, 0)),
            pl.BlockSpec((N_TOK, NUM_EXPERTS), lambda b: (0, 0)),
            pl.BlockSpec((BT, NUM_EXPERTS), lambda b: (b, 0)),
            pl.BlockSpec((BT, NUM_EXPERTS), lambda b: (b, 0)),
        ],
        out_specs=[
            pl.BlockSpec((BT, 1), lambda b: (b, 0)),
            pl.BlockSpec((BT, 1), lambda b: (b, 0)),
            pl.BlockSpec((MAX_TILES, 1), lambda b: (0, 0)),
        ],
        out_shape=[
            jax.ShapeDtypeStruct((N_TOK, 1), jnp.int32),
            jax.ShapeDtypeStruct((N_TOK, 1), jnp.int32),
            jax.ShapeDtypeStruct((MAX_TILES, 1), jnp.int32),
        ],
    )(oh0, oh1, oh0, oh1)


def _scatter_body(x_hbm, pos0_hbm, pos1_hbm, xg_hbm, pidx0, pidx1, xv, sem):
    c = lax.axis_index("c")
    s = lax.axis_index("s")
    g = c * 16 + s
    base = g * TPW
    pltpu.sync_copy(pos0_hbm.at[pl.ds(base, TPW)], pidx0)
    pltpu.sync_copy(pos1_hbm.at[pl.ds(base, TPW)], pidx1)
    pltpu.sync_copy(x_hbm.at[pl.ds(base, TPW), :], xv)
    pltpu.async_copy(xv, xg_hbm.at[pidx0], sem).wait()
    pltpu.async_copy(xv, xg_hbm.at[pidx1], sem).wait()


@functools.partial(
    pl.kernel, mesh=_mesh,
    out_type=[jax.ShapeDtypeStruct((ROWS, N_EMBD), jnp.float32)],
    scratch_types=[
        pltpu.VMEM((TPW,), jnp.int32),
        pltpu.VMEM((TPW,), jnp.int32),
        pltpu.VMEM((TPW, N_EMBD), jnp.float32),
        pltpu.SemaphoreType.DMA,
    ],
)
def _scatter(x_hbm, pos0_hbm, pos1_hbm, xg_hbm, *scratch):
    _scatter_body(x_hbm, pos0_hbm, pos1_hbm, xg_hbm, *scratch)


def _gemm_body(te_ref, xg_ref, wfca_ref, wfcb_ref, wpra_ref, wprb_ref,
               yo_ref):
    xb = xg_ref[...].astype(jnp.bfloat16)
    ha = lax.dot_general(
        xb, wfca_ref[0].astype(jnp.bfloat16), (((1,), (1,)), ((), ())),
        preferred_element_type=jnp.float32)
    hb = lax.dot_general(
        xb, wfcb_ref[0].astype(jnp.bfloat16), (((1,), (1,)), ((), ())),
        preferred_element_type=jnp.float32)
    ha = jnp.square(jnp.maximum(ha, 0.0)).astype(jnp.bfloat16)
    hb = jnp.square(jnp.maximum(hb, 0.0)).astype(jnp.bfloat16)
    yo_ref[...] = (
        lax.dot_general(
            ha, wpra_ref[0].astype(jnp.bfloat16), (((1,), (1,)), ((), ())),
            preferred_element_type=jnp.float32)
        + lax.dot_general(
            hb, wprb_ref[0].astype(jnp.bfloat16), (((1,), (1,)), ((), ())),
            preferred_element_type=jnp.float32))


def _gemm(te, xg, W_fc, W_proj):
    HH = EXPERT_DIM // 2
    grid_spec = pltpu.PrefetchScalarGridSpec(
        num_scalar_prefetch=1,
        grid=(MAX_TILES,),
        in_specs=[
            pl.BlockSpec((BT, N_EMBD), lambda t, te_ref: (t, 0)),
            pl.BlockSpec((1, HH, N_EMBD),
                         lambda t, te_ref: (te_ref[t], 0, 0)),
            pl.BlockSpec((1, HH, N_EMBD),
                         lambda t, te_ref: (te_ref[t], 1, 0)),
            pl.BlockSpec((1, N_EMBD, HH),
                         lambda t, te_ref: (te_ref[t], 0, 0)),
            pl.BlockSpec((1, N_EMBD, HH),
                         lambda t, te_ref: (te_ref[t], 0, 1)),
        ],
        out_specs=pl.BlockSpec((BT, N_EMBD), lambda t, te_ref: (t, 0)),
    )
    return pl.pallas_call(
        _gemm_body,
        grid_spec=grid_spec,
        out_shape=jax.ShapeDtypeStruct((ROWS, N_EMBD), jnp.float32),
    )(te, xg, W_fc, W_fc, W_proj, W_proj)


def _gather_body(yo_hbm, pos0_hbm, pos1_hbm, y0_hbm, y1_hbm,
                 pidx0, pidx1, buf0, buf1, sem):
    c = lax.axis_index("c")
    s = lax.axis_index("s")
    g = c * 16 + s
    base = g * TPW
    pltpu.sync_copy(pos0_hbm.at[pl.ds(base, TPW)], pidx0)
    pltpu.sync_copy(pos1_hbm.at[pl.ds(base, TPW)], pidx1)
    pltpu.async_copy(yo_hbm.at[pidx0], buf0, sem).wait()
    pltpu.async_copy(yo_hbm.at[pidx1], buf1, sem).wait()
    pltpu.sync_copy(buf0, y0_hbm.at[pl.ds(base, TPW), :])
    pltpu.sync_copy(buf1, y1_hbm.at[pl.ds(base, TPW), :])


@functools.partial(
    pl.kernel, mesh=_mesh,
    out_type=[
        jax.ShapeDtypeStruct((N_TOK, N_EMBD), jnp.float32),
        jax.ShapeDtypeStruct((N_TOK, N_EMBD), jnp.float32),
    ],
    scratch_types=[
        pltpu.VMEM((TPW,), jnp.int32),
        pltpu.VMEM((TPW,), jnp.int32),
        pltpu.VMEM((TPW, N_EMBD), jnp.float32),
        pltpu.VMEM((TPW, N_EMBD), jnp.float32),
        pltpu.SemaphoreType.DMA,
    ],
)
def _gather(yo_hbm, pos0_hbm, pos1_hbm, y0_hbm, y1_hbm, *scratch):
    _gather_body(yo_hbm, pos0_hbm, pos1_hbm, y0_hbm, y1_hbm, *scratch)


def _combine_body(y0_ref, y1_ref, w0_ref, w1_ref, out_ref):
    out_ref[...] = (y0_ref[...] * w0_ref[...]
                    + y1_ref[...] * w1_ref[...])


def _combine(y0, y1, w0, w1):
    return pl.pallas_call(
        _combine_body,
        out_shape=jax.ShapeDtypeStruct((N_TOK, N_EMBD), jnp.float32),
    )(y0, y1, w0, w1)


def kernel(x, Wg, W_fc, W_proj):
    B, T, D = x.shape
    flat_x = x.reshape(N_TOK, D)
    w0, w1, oh0, oh1 = _router(flat_x, Wg)
    pos0, pos1, te = _plan(oh0, oh1)
    xg, = _scatter(flat_x, pos0.reshape(N_TOK), pos1.reshape(N_TOK))
    yo = _gemm(te.reshape(MAX_TILES), xg, W_fc, W_proj)
    y0, y1 = _gather(yo, pos0.reshape(N_TOK), pos1.reshape(N_TOK))
    out = _combine(y0, y1, w0, w1)
    return out.reshape(B, T, D)


# speedup vs baseline: 2.5575x; 1.0113x over previous
import functools

import jax
import jax.numpy as jnp
from jax import lax
from jax.experimental import pallas as pl
from jax.experimental.pallas import tpu as pltpu
from jax.experimental.pallas import tpu_sc as plsc

N_EMBD = 768
EXPERT_DIM = 1536
NUM_EXPERTS = 64
TOP_K = 2

N_TOK = 2048
BT = 128
MAX_TILES = 96
ROWS = MAX_TILES * BT
NW = 32
TPW = N_TOK // NW
LANES = 16

_mesh = plsc.VectorSubcoreMesh(core_axis_name="c", subcore_axis_name="s")


def _router_body(x_ref, wg_ref, w0_ref, w1_ref, oh0_ref, oh1_ref):
    x = x_ref[...]
    wg = wg_ref[...]
    logits = lax.dot_general(
        x.astype(jnp.bfloat16), wg.astype(jnp.bfloat16),
        (((1,), (1,)), ((), ())),
        preferred_element_type=jnp.float32)
    p = jax.nn.softmax(logits, axis=-1)
    m1 = jnp.max(p, axis=-1)
    a1 = jnp.argmax(p, axis=-1).astype(jnp.int32)
    e_iota = lax.broadcasted_iota(jnp.int32, p.shape, 1)
    p2 = jnp.where(e_iota == a1[:, None], -jnp.inf, p)
    m2 = jnp.max(p2, axis=-1)
    a2 = jnp.argmax(p2, axis=-1).astype(jnp.int32)
    denom = m1 + m2 + 1e-8
    w0_ref[...] = (m1 / denom)[:, None]
    w1_ref[...] = (m2 / denom)[:, None]
    oh0_ref[...] = (e_iota == a1[:, None]).astype(jnp.bfloat16)
    oh1_ref[...] = (e_iota == a2[:, None]).astype(jnp.bfloat16)


def _router(flat_x, Wg):
    return pl.pallas_call(
        _router_body,
        out_shape=[
            jax.ShapeDtypeStruct((N_TOK, 1), jnp.float32),
            jax.ShapeDtypeStruct((N_TOK, 1), jnp.float32),
            jax.ShapeDtypeStruct((N_TOK, NUM_EXPERTS), jnp.bfloat16),
            jax.ShapeDtypeStruct((N_TOK, NUM_EXPERTS), jnp.bfloat16),
        ],
    )(flat_x, Wg)


def _plan_body(oh0_ref, oh1_ref, oh0b_ref, oh1b_ref,
               pos0_ref, pos1_ref, te_ref, txg_ref, tout_ref):
    b = pl.program_id(0)
    oh0 = oh0_ref[...]
    oh1 = oh1_ref[...]
    tot0 = jnp.sum(oh0.astype(jnp.float32), axis=0)
    cnt = tot0 + jnp.sum(oh1.astype(jnp.float32), axis=0)
    pc = jnp.floor((cnt + (BT - 1)) / BT) * BT
    ei = lax.broadcasted_iota(jnp.int32, (NUM_EXPERTS, NUM_EXPERTS), 0)
    ej = lax.broadcasted_iota(jnp.int32, (NUM_EXPERTS, NUM_EXPERTS), 1)
    trilE = (ej < ei).astype(jnp.bfloat16)
    base = lax.dot_general(
        trilE, pc.astype(jnp.bfloat16), (((1,), (0,)), ((), ())),
        preferred_element_type=jnp.float32)

    ri = lax.broadcasted_iota(jnp.int32, (BT, N_TOK), 0)
    cj = lax.broadcasted_iota(jnp.int32, (BT, N_TOK), 1)
    tril = (cj < b * BT + ri).astype(jnp.bfloat16)
    c0 = lax.dot_general(
        tril, oh0, (((1,), (0,)), ((), ())),
        preferred_element_type=jnp.float32)
    c1 = lax.dot_general(
        tril, oh1, (((1,), (0,)), ((), ())),
        preferred_element_type=jnp.float32)
    ohb0 = oh0b_ref[...].astype(jnp.float32)
    ohb1 = oh1b_ref[...].astype(jnp.float32)
    pos0 = jnp.sum((c0 + base[None, :]) * ohb0, axis=1)
    pos1 = jnp.sum((c1 + tot0[None, :] + base[None, :]) * ohb1, axis=1)
    pos0_ref[...] = pos0.astype(jnp.int32)[:, None]
    pos1_ref[...] = pos1.astype(jnp.int32)[:, None]

    @pl.when(b == 0)
    def _():
        bt_tile = base / BT
        ti = lax.broadcasted_iota(
            jnp.int32, (MAX_TILES, NUM_EXPERTS), 0).astype(jnp.float32)
        te = jnp.sum((bt_tile[None, :] <= ti).astype(jnp.float32),
                     axis=1) - 1.0
        te_ref[...] = te.astype(jnp.int32)[:, None]
        tt = jnp.sum(pc) / BT
        tf = lax.broadcasted_iota(
            jnp.int32, (MAX_TILES, 1), 0).astype(jnp.float32)
        valid = tf < tt
        txg_ref[...] = jnp.where(valid, tf, tt - 1.0).astype(jnp.int32)
        tout_ref[...] = jnp.where(
            valid, tf, float(MAX_TILES - 1)).astype(jnp.int32)


def _plan(oh0, oh1):
    nblk = N_TOK // BT
    return pl.pallas_call(
        _plan_body,
        grid=(nblk,),
        in_specs=[
            pl.BlockSpec((N_TOK, NUM_EXPERTS), lambda b: (0, 0)),
            pl.BlockSpec((N_TOK, NUM_EXPERTS), lambda b: (0, 0)),
            pl.BlockSpec((BT, NUM_EXPERTS), lambda b: (b, 0)),
            pl.BlockSpec((BT, NUM_EXPERTS), lambda b: (b, 0)),
        ],
        out_specs=[
            pl.BlockSpec((BT, 1), lambda b: (b, 0)),
            pl.BlockSpec((BT, 1), lambda b: (b, 0)),
            pl.BlockSpec((MAX_TILES, 1), lambda b: (0, 0)),
            pl.BlockSpec((MAX_TILES, 1), lambda b: (0, 0)),
            pl.BlockSpec((MAX_TILES, 1), lambda b: (0, 0)),
        ],
        out_shape=[
            jax.ShapeDtypeStruct((N_TOK, 1), jnp.int32),
            jax.ShapeDtypeStruct((N_TOK, 1), jnp.int32),
            jax.ShapeDtypeStruct((MAX_TILES, 1), jnp.int32),
            jax.ShapeDtypeStruct((MAX_TILES, 1), jnp.int32),
            jax.ShapeDtypeStruct((MAX_TILES, 1), jnp.int32),
        ],
    )(oh0, oh1, oh0, oh1)


def _scatter_body(x_hbm, pos0_hbm, pos1_hbm, xg_hbm, pidx0, pidx1, xv, sem):
    c = lax.axis_index("c")
    s = lax.axis_index("s")
    g = c * 16 + s
    base = g * TPW
    pltpu.sync_copy(pos0_hbm.at[pl.ds(base, TPW)], pidx0)
    pltpu.sync_copy(pos1_hbm.at[pl.ds(base, TPW)], pidx1)
    pltpu.sync_copy(x_hbm.at[pl.ds(base, TPW), :], xv)
    pltpu.async_copy(xv, xg_hbm.at[pidx0], sem).wait()
    pltpu.async_copy(xv, xg_hbm.at[pidx1], sem).wait()


@functools.partial(
    pl.kernel, mesh=_mesh,
    out_type=[jax.ShapeDtypeStruct((ROWS, N_EMBD), jnp.float32)],
    scratch_types=[
        pltpu.VMEM((TPW,), jnp.int32),
        pltpu.VMEM((TPW,), jnp.int32),
        pltpu.VMEM((TPW, N_EMBD), jnp.float32),
        pltpu.SemaphoreType.DMA,
    ],
)
def _scatter(x_hbm, pos0_hbm, pos1_hbm, xg_hbm, *scratch):
    _scatter_body(x_hbm, pos0_hbm, pos1_hbm, xg_hbm, *scratch)


def _gemm_body(te_ref, txg_ref, tout_ref, xg_ref, wfc_ref, wpr_ref,
               yo_ref):
    xb = xg_ref[...].astype(jnp.bfloat16)
    wfc = wfc_ref[0].astype(jnp.bfloat16)
    h = lax.dot_general(
        xb, wfc, (((1,), (1,)), ((), ())),
        preferred_element_type=jnp.float32)
    h = jnp.square(jnp.maximum(h, 0.0)).astype(jnp.bfloat16)
    wpr = wpr_ref[0].astype(jnp.bfloat16)
    yo_ref[...] = lax.dot_general(
        h, wpr, (((1,), (1,)), ((), ())),
        preferred_element_type=jnp.float32)


def _gemm(te, txg, tout, xg, W_fc, W_proj):
    grid_spec = pltpu.PrefetchScalarGridSpec(
        num_scalar_prefetch=3,
        grid=(MAX_TILES,),
        in_specs=[
            pl.BlockSpec((BT, N_EMBD), lambda t, te, txg, tout: (txg[t], 0)),
            pl.BlockSpec((1, EXPERT_DIM, N_EMBD),
                         lambda t, te, txg, tout: (te[t], 0, 0)),
            pl.BlockSpec((1, N_EMBD, EXPERT_DIM),
                         lambda t, te, txg, tout: (te[t], 0, 0)),
        ],
        out_specs=pl.BlockSpec(
            (BT, N_EMBD), lambda t, te, txg, tout: (tout[t], 0)),
    )
    return pl.pallas_call(
        _gemm_body,
        grid_spec=grid_spec,
        out_shape=jax.ShapeDtypeStruct((ROWS, N_EMBD), jnp.float32),
    )(te, txg, tout, xg, W_fc, W_proj)


def _gather_body(yo_hbm, pos0_hbm, pos1_hbm, y0_hbm, y1_hbm,
                 pidx0, pidx1, buf0, buf1, sem):
    c = lax.axis_index("c")
    s = lax.axis_index("s")
    g = c * 16 + s
    base = g * TPW
    pltpu.sync_copy(pos0_hbm.at[pl.ds(base, TPW)], pidx0)
    pltpu.sync_copy(pos1_hbm.at[pl.ds(base, TPW)], pidx1)
    pltpu.async_copy(yo_hbm.at[pidx0], buf0, sem).wait()
    pltpu.async_copy(yo_hbm.at[pidx1], buf1, sem).wait()
    pltpu.sync_copy(buf0, y0_hbm.at[pl.ds(base, TPW), :])
    pltpu.sync_copy(buf1, y1_hbm.at[pl.ds(base, TPW), :])


@functools.partial(
    pl.kernel, mesh=_mesh,
    out_type=[
        jax.ShapeDtypeStruct((N_TOK, N_EMBD), jnp.float32),
        jax.ShapeDtypeStruct((N_TOK, N_EMBD), jnp.float32),
    ],
    scratch_types=[
        pltpu.VMEM((TPW,), jnp.int32),
        pltpu.VMEM((TPW,), jnp.int32),
        pltpu.VMEM((TPW, N_EMBD), jnp.float32),
        pltpu.VMEM((TPW, N_EMBD), jnp.float32),
        pltpu.SemaphoreType.DMA,
    ],
)
def _gather(yo_hbm, pos0_hbm, pos1_hbm, y0_hbm, y1_hbm, *scratch):
    _gather_body(yo_hbm, pos0_hbm, pos1_hbm, y0_hbm, y1_hbm, *scratch)


def _combine_body(y0_ref, y1_ref, w0_ref, w1_ref, out_ref):
    out_ref[...] = (y0_ref[...] * w0_ref[...]
                    + y1_ref[...] * w1_ref[...])


def _combine(y0, y1, w0, w1):
    return pl.pallas_call(
        _combine_body,
        out_shape=jax.ShapeDtypeStruct((N_TOK, N_EMBD), jnp.float32),
    )(y0, y1, w0, w1)


def kernel(x, Wg, W_fc, W_proj):
    B, T, D = x.shape
    flat_x = x.reshape(N_TOK, D)
    w0, w1, oh0, oh1 = _router(flat_x, Wg)
    pos0, pos1, te, txg, tout = _plan(oh0, oh1)
    xg, = _scatter(flat_x, pos0.reshape(N_TOK), pos1.reshape(N_TOK))
    yo = _gemm(te.reshape(MAX_TILES), txg.reshape(MAX_TILES),
               tout.reshape(MAX_TILES), xg, W_fc, W_proj)
    y0, y1 = _gather(yo, pos0.reshape(N_TOK), pos1.reshape(N_TOK))
    out = _combine(y0, y1, w0, w1)
    return out.reshape(B, T, D)
